# Initial kernel scaffold; baseline (speedup 1.0000x reference)
import jax
import jax.numpy as jnp
from jax.experimental import pallas as pl


def kernel(distance_matrix):
    raise NotImplementedError("write your pallas kernel here")


# speedup vs baseline: 27.5263x; 27.5263x over previous
import functools

import numpy as np
import jax
import jax.numpy as jnp
from jax import lax
from jax.experimental import pallas as pl
from jax.experimental.pallas import tpu as pltpu
from jax.experimental.pallas import tpu_sc as plsc

BATCH = 4096
KCLS = 8
MARGIN = 0.3
LANES = 16
NC = 2
NS = 16
NW = NC * NS
ROWS_PER_W = BATCH // NW
RSLAB = 8
NSLAB = ROWS_PER_W // RSLAB
CHUNKS = BATCH // LANES
BIG = np.float32(3.0e38)


def _sc_body(d_hbm, out_hbm, buf, accv):
    c = lax.axis_index("c")
    s = lax.axis_index("s")
    wid = s * NC + c
    row0 = wid * ROWS_PER_W
    lane = lax.iota(jnp.int32, LANES)

    def slab_body(t, total):
        r0 = row0 + t * RSLAB
        pltpu.sync_copy(d_hbm.at[pl.ds(r0, RSLAB)], buf)
        for r in range(RSLAB):
            i = r0 + r
            cs = i // LANES
            off = ((i // KCLS) % 2) * KCLS
            dlane = i % LANES
            v = buf[r, pl.ds(cs * LANES, LANES)]
            in_block = (lane >= off) & (lane < off + KCLS)
            mask_pos = in_block & (lane != dlane)
            pos = jnp.max(jnp.where(mask_pos, v, -BIG))
            buf[r, pl.ds(cs * LANES, LANES)] = jnp.where(mask_pos, BIG, v)

            def chunk_step(j, carry):
                m1, m2 = carry
                w = buf[r, pl.ds(j * LANES, LANES)]
                m2 = jnp.minimum(m2, jnp.maximum(m1, w))
                m1 = jnp.minimum(m1, w)
                return m1, m2

            init = (jnp.full((LANES,), BIG, jnp.float32),
                    jnp.full((LANES,), BIG, jnp.float32))
            m1, m2 = lax.fori_loop(0, CHUNKS, chunk_step, init)

            g1 = jnp.min(m1)
            first = plsc.all_reduce_ffs(m1 == g1)
            u = jnp.where(lane == first, BIG, m1)
            g2 = jnp.minimum(jnp.min(u), jnp.min(m2))
            total = total + jnp.maximum(jnp.maximum(pos, 0.0) - g2 + MARGIN, 0.0)
        return total

    total = lax.fori_loop(0, NSLAB, slab_body, jnp.float32(0.0))
    accv[...] = jnp.where(lane == 0, total, jnp.float32(0.0))
    pltpu.sync_copy(accv, out_hbm.at[wid])


@functools.partial(
    pl.kernel,
    out_type=jax.ShapeDtypeStruct((NW, LANES), jnp.float32),
    mesh=plsc.VectorSubcoreMesh(core_axis_name="c", subcore_axis_name="s"),
    scratch_types=[
        pltpu.VMEM((RSLAB, BATCH), jnp.float32),
        pltpu.VMEM((LANES,), jnp.float32),
    ],
    compiler_params=pltpu.CompilerParams(needs_layout_passes=False),
)
def _sc_triplet(d_hbm, out_hbm, buf, accv):
    _sc_body(d_hbm, out_hbm, buf, accv)


def kernel(distance_matrix):
    partials = _sc_triplet(distance_matrix)
    return jnp.sum(partials) / jnp.float32(BATCH)


# speedup vs baseline: 53.7119x; 1.9513x over previous
import functools

import numpy as np
import jax
import jax.numpy as jnp
from jax import lax
from jax.experimental import pallas as pl
from jax.experimental.pallas import tpu as pltpu
from jax.experimental.pallas import tpu_sc as plsc

BATCH = 4096
KCLS = 8
MARGIN = 0.3
LANES = 16
NC = 2
NS = 16
NW = NC * NS
ROWS_PER_W = BATCH // NW
RSLAB = 8
NSLAB = ROWS_PER_W // RSLAB
CHUNKS = BATCH // LANES
UNROLL = 4
SUB = CHUNKS // UNROLL
BIG = np.float32(3.0e38)


def _sc_body(d_hbm, out_hbm, buf, accv):
    c = lax.axis_index("c")
    s = lax.axis_index("s")
    wid = s * NC + c
    row0 = wid * ROWS_PER_W
    lane = lax.iota(jnp.int32, LANES)

    def slab_body(t, total):
        r0 = row0 + t * RSLAB
        pltpu.sync_copy(d_hbm.at[pl.ds(r0, RSLAB)], buf)
        for r in range(RSLAB):
            i = r0 + r
            cs = i // LANES
            off = ((i // KCLS) % 2) * KCLS
            dlane = i % LANES
            v = buf[r, pl.ds(cs * LANES, LANES)]
            in_block = (lane >= off) & (lane < off + KCLS)
            mask_pos = in_block & (lane != dlane)
            pos = jnp.max(jnp.where(mask_pos, v, -BIG))
            buf[r, pl.ds(cs * LANES, LANES)] = jnp.where(mask_pos, BIG, v)

            def chunk_step(j, carry):
                new = []
                for k in range(UNROLL):
                    m1, m2 = carry[2 * k], carry[2 * k + 1]
                    w = buf[r, pl.ds((j + k * SUB) * LANES, LANES)]
                    new.append(jnp.minimum(m1, w))
                    new.append(jnp.minimum(m2, jnp.maximum(m1, w)))
                return tuple(new)

            init = tuple(jnp.full((LANES,), BIG, jnp.float32)
                         for _ in range(2 * UNROLL))
            acc = lax.fori_loop(0, SUB, chunk_step, init)
            m1, m2 = acc[0], acc[1]
            for k in range(1, UNROLL):
                y1, y2 = acc[2 * k], acc[2 * k + 1]
                m2 = jnp.minimum(jnp.maximum(m1, y1), jnp.minimum(m2, y2))
                m1 = jnp.minimum(m1, y1)

            g1 = jnp.min(m1)
            first = plsc.all_reduce_ffs(m1 == g1)
            u = jnp.where(lane == first, BIG, m1)
            g2 = jnp.minimum(jnp.min(u), jnp.min(m2))
            total = total + jnp.maximum(jnp.maximum(pos, 0.0) - g2 + MARGIN, 0.0)
        return total

    total = lax.fori_loop(0, NSLAB, slab_body, jnp.float32(0.0))
    accv[...] = jnp.where(lane == 0, total, jnp.float32(0.0))
    pltpu.sync_copy(accv, out_hbm.at[wid])


@functools.partial(
    pl.kernel,
    out_type=jax.ShapeDtypeStruct((NW, LANES), jnp.float32),
    mesh=plsc.VectorSubcoreMesh(core_axis_name="c", subcore_axis_name="s"),
    scratch_types=[
        pltpu.VMEM((RSLAB, BATCH), jnp.float32),
        pltpu.VMEM((LANES,), jnp.float32),
    ],
    compiler_params=pltpu.CompilerParams(needs_layout_passes=False),
)
def _sc_triplet(d_hbm, out_hbm, buf, accv):
    _sc_body(d_hbm, out_hbm, buf, accv)


def kernel(distance_matrix):
    partials = _sc_triplet(distance_matrix)
    return jnp.sum(partials) / jnp.float32(BATCH)


# speedup vs baseline: 80.3165x; 1.4953x over previous
import functools

import numpy as np
import jax
import jax.numpy as jnp
from jax import lax
from jax.experimental import pallas as pl
from jax.experimental.pallas import tpu as pltpu
from jax.experimental.pallas import tpu_sc as plsc

BATCH = 4096
KCLS = 8
MARGIN = 0.3
LANES = 16
NC = 2
NS = 16
NW = NC * NS
ROWS_PER_W = BATCH // NW
RSLAB = 8
NSLAB = ROWS_PER_W // RSLAB
CHUNKS = BATCH // LANES
UNROLL = 4
SUB = CHUNKS // UNROLL
BIG = np.float32(3.0e38)


def _sc_body(d_hbm, out_hbm, buf0, buf1, accv, sem0, sem1):
    c = lax.axis_index("c")
    s = lax.axis_index("s")
    wid = s * NC + c
    row0 = wid * ROWS_PER_W
    lane = lax.iota(jnp.int32, LANES)
    bufs = (buf0, buf1)
    sems = (sem0, sem1)

    def start(t, b):
        pltpu.async_copy(d_hbm.at[pl.ds(row0 + t * RSLAB, RSLAB)],
                         bufs[b], sems[b])

    def wait(t, b):
        pltpu.make_async_copy(d_hbm.at[pl.ds(row0 + t * RSLAB, RSLAB)],
                              bufs[b], sems[b]).wait()

    def process_slab(buf, r0, total):
        for r in range(RSLAB):
            i = r0 + r
            cs = i // LANES
            off = ((i // KCLS) % 2) * KCLS
            dlane = i % LANES
            v = buf[r, pl.ds(cs * LANES, LANES)]
            in_block = (lane >= off) & (lane < off + KCLS)
            mask_pos = in_block & (lane != dlane)
            pos = jnp.max(jnp.where(mask_pos, v, -BIG))
            buf[r, pl.ds(cs * LANES, LANES)] = jnp.where(mask_pos, BIG, v)

            def chunk_step(j, carry):
                new = []
                for k in range(UNROLL):
                    m1, m2 = carry[2 * k], carry[2 * k + 1]
                    w = buf[r, pl.ds((j + k * SUB) * LANES, LANES)]
                    new.append(jnp.minimum(m1, w))
                    new.append(jnp.minimum(m2, jnp.maximum(m1, w)))
                return tuple(new)

            init = tuple(jnp.full((LANES,), BIG, jnp.float32)
                         for _ in range(2 * UNROLL))
            acc = lax.fori_loop(0, SUB, chunk_step, init)
            m1, m2 = acc[0], acc[1]
            for k in range(1, UNROLL):
                y1, y2 = acc[2 * k], acc[2 * k + 1]
                m2 = jnp.minimum(jnp.maximum(m1, y1), jnp.minimum(m2, y2))
                m1 = jnp.minimum(m1, y1)

            g1 = jnp.min(m1)
            first = plsc.all_reduce_ffs(m1 == g1)
            u = jnp.where(lane == first, BIG, m1)
            g2 = jnp.minimum(jnp.min(u), jnp.min(m2))
            total = total + jnp.maximum(jnp.maximum(pos, 0.0) - g2 + MARGIN, 0.0)
        return total

    start(0, 0)

    def outer(h, total):
        t0 = 2 * h
        wait(t0, 0)
        start(t0 + 1, 1)
        total = process_slab(buf0, row0 + t0 * RSLAB, total)
        wait(t0 + 1, 1)

        @pl.when(t0 + 2 < NSLAB)
        def _():
            start(t0 + 2, 0)

        total = process_slab(buf1, row0 + (t0 + 1) * RSLAB, total)
        return total

    total = lax.fori_loop(0, NSLAB // 2, outer, jnp.float32(0.0))
    accv[...] = jnp.where(lane == 0, total, jnp.float32(0.0))
    pltpu.sync_copy(accv, out_hbm.at[wid])


@functools.partial(
    pl.kernel,
    out_type=jax.ShapeDtypeStruct((NW, LANES), jnp.float32),
    mesh=plsc.VectorSubcoreMesh(core_axis_name="c", subcore_axis_name="s"),
    scratch_types=[
        pltpu.VMEM((RSLAB, BATCH), jnp.float32),
        pltpu.VMEM((RSLAB, BATCH), jnp.float32),
        pltpu.VMEM((LANES,), jnp.float32),
        pltpu.SemaphoreType.DMA,
        pltpu.SemaphoreType.DMA,
    ],
    compiler_params=pltpu.CompilerParams(needs_layout_passes=False),
)
def _sc_triplet(d_hbm, out_hbm, buf0, buf1, accv, sem0, sem1):
    _sc_body(d_hbm, out_hbm, buf0, buf1, accv, sem0, sem1)


def kernel(distance_matrix):
    partials = _sc_triplet(distance_matrix)
    return jnp.sum(partials) / jnp.float32(BATCH)
